# baseline (speedup 1.0000x reference)
import functools

import jax
import jax.numpy as jnp
from jax import lax
from jax.experimental import pallas as pl
from jax.experimental.pallas import tpu as pltpu
from jax.experimental.pallas import tpu_sc as plsc

N = 10000
E = 320000
D = 128
G = 256

NC = 2
NS = 16
CH = 128
NCHUNKS = E // CH
COMMON = NCHUNKS // (NC * NS)
EXTRA = NCHUNKS - COMMON * NC * NS
NBUF = 3
RPT = 624


def _sc_body(x_hbm, ei_hbm, agg_hbm, *rest):
    ibuf = rest[:2 * NBUF]
    isem = rest[2 * NBUF:4 * NBUF]
    rows = rest[4 * NBUF:5 * NBUF]
    gsem = rest[5 * NBUF:6 * NBUF]
    ssem = rest[6 * NBUF:7 * NBUF]
    acc = rest[7 * NBUF]
    c = lax.axis_index("c")
    s = lax.axis_index("s")
    t = c * NS + s
    base = t * COMMON

    def _e(j):
        return pl.multiple_of((base + j) * CH, CH)

    def _i(j, p, b):
        return pltpu.make_async_copy(ei_hbm.at[:, pl.ds(_e(j), CH)],
                                     ibuf[p * NBUF + b], isem[p * NBUF + b])

    def _g(j, p, b):
        return pltpu.make_async_copy(x_hbm.at[ibuf[p * NBUF + b].at[0]],
                                     rows[b], gsem[b])

    def _s(j, p, b):
        return pltpu.make_async_copy(rows[b], acc.at[ibuf[p * NBUF + b].at[1]],
                                     ssem[b])

    for b in range(NBUF):
        _i(b, 0, b).start()

    def _zfill(i, carry):
        for b in range(NBUF):
            for j in range(D // 16):
                rows[b][i, pl.ds(j * 16, 16)] = jnp.zeros((16,), jnp.float32)
        return carry
    lax.fori_loop(0, CH, _zfill, 0)
    row0 = s * RPT
    for k in range(RPT // CH):
        pltpu.sync_copy(rows[k % NBUF], acc.at[pl.ds(row0 + k * CH, CH)])
    pltpu.sync_copy(rows[0].at[pl.ds(0, RPT - (RPT // CH) * CH)],
                    acc.at[pl.ds(row0 + (RPT // CH) * CH,
                                 RPT - (RPT // CH) * CH)])

    @pl.when(s == NS - 1)
    def _():
        left = N - NS * RPT
        pltpu.sync_copy(rows[0].at[pl.ds(0, left)],
                        acc.at[pl.ds(NS * RPT, left)])

    for b in range(NBUF):
        _i(b, 0, b).wait()
        _g(b, 0, b).start()
    plsc.subcore_barrier()

    def _one_group(g, par, prefetch):
        j0 = g * NBUF
        for b in range(NBUF):
            _g(j0 + b, par, b).wait()
            _s(j0 + b, par, b).start(add=True)
            if prefetch:
                _i(j0 + NBUF + b, 1 - par, b).start()
        for b in range(NBUF):
            _s(j0 + b, par, b).wait()
            if prefetch:
                _i(j0 + NBUF + b, 1 - par, b).wait()
                _g(j0 + NBUF + b, 1 - par, b).start()

    def _pair(k2, carry):
        _one_group(2 * k2, 0, True)
        _one_group(2 * k2 + 1, 1, True)
        return carry
    NG = COMMON // NBUF
    lax.fori_loop(0, NG // 2 - 1, _pair, 0)
    _one_group(NG - 2, 0, True)
    _one_group(NG - 1, 1, False)

    @pl.when(s < EXTRA // NC)
    def _():
        off = pl.multiple_of(
            (NC * NS * COMMON) * CH + (c * (EXTRA // NC) + s) * CH, CH)
        pltpu.sync_copy(ei_hbm.at[:, pl.ds(off, CH)], ibuf[0])
        _g(0, 0, 0).start()
        _g(0, 0, 0).wait()
        pltpu.sync_copy(rows[0], acc.at[ibuf[0].at[1]], add=True)

    plsc.subcore_barrier()

    pltpu.sync_copy(acc.at[pl.ds(row0, RPT)], agg_hbm.at[c, pl.ds(row0, RPT)])

    @pl.when(s == NS - 1)
    def _():
        left = N - NS * RPT
        pltpu.sync_copy(acc.at[pl.ds(NS * RPT, left)],
                        agg_hbm.at[c, pl.ds(NS * RPT, left)])


@jax.jit
def _sc_scatter(x, edge_index):
    mesh = plsc.VectorSubcoreMesh(core_axis_name="c", subcore_axis_name="s")
    scratch = [pltpu.VMEM((2, CH), jnp.int32) for _ in range(2 * NBUF)]
    scratch += [pltpu.SemaphoreType.DMA for _ in range(2 * NBUF)]
    scratch += [pltpu.VMEM((CH, D), jnp.float32) for _ in range(NBUF)]
    scratch += [pltpu.SemaphoreType.DMA for _ in range(2 * NBUF)]
    scratch += [pltpu.MemorySpace.VMEM_SHARED((N, D), jnp.float32)]
    return pl.kernel(
        _sc_body,
        out_type=jax.ShapeDtypeStruct((NC, N, D), jnp.float32),
        mesh=mesh,
        scratch_types=scratch,
    )(x, edge_index)


BLK = 2000
NBLK = N // BLK


def _tc_body(agg_ref, x_ref, batch_ref, wn_ref, wr_ref, b_ref, out_ref):
    i = pl.program_id(0)
    a = (agg_ref[0] + agg_ref[1]).astype(jnp.bfloat16)
    h = jnp.dot(a, wn_ref[...].astype(jnp.bfloat16),
                preferred_element_type=jnp.float32)
    h = h + jnp.dot(x_ref[...].astype(jnp.bfloat16),
                    wr_ref[...].astype(jnp.bfloat16),
                    preferred_element_type=jnp.float32)
    h = jnp.maximum(h + b_ref[...], 0.0)
    bt = batch_ref[0, 0, :]
    gid = lax.broadcasted_iota(jnp.int32, (G, BLK), 0)
    onehot = jnp.where(gid == bt[None, :], 1.0, 0.0).astype(jnp.bfloat16)
    p = jnp.dot(onehot, h.astype(jnp.bfloat16),
                preferred_element_type=jnp.float32)

    @pl.when(i == 0)
    def _():
        out_ref[...] = p

    @pl.when(i > 0)
    def _():
        out_ref[...] += p


@jax.jit
def _tc_combine(agg2, x, batch3, W_nbr, W_root, b2):
    return pl.pallas_call(
        _tc_body,
        grid=(NBLK,),
        in_specs=[
            pl.BlockSpec((NC, BLK, D), lambda i: (0, i, 0)),
            pl.BlockSpec((BLK, D), lambda i: (i, 0)),
            pl.BlockSpec((1, 1, BLK), lambda i: (i, 0, 0)),
            pl.BlockSpec((D, D), lambda i: (0, 0)),
            pl.BlockSpec((D, D), lambda i: (0, 0)),
            pl.BlockSpec((1, D), lambda i: (0, 0)),
        ],
        out_specs=pl.BlockSpec((G, D), lambda i: (0, 0)),
        out_shape=jax.ShapeDtypeStruct((G, D), jnp.float32),
    )(agg2, x, batch3, W_nbr, W_root, b2)


def kernel(x, edge_index, batch, W_nbr, W_root, b):
    agg2 = _sc_scatter(x, edge_index)
    batch3 = batch.reshape(NBLK, 1, BLK)
    b2 = b.reshape(1, D)
    return _tc_combine(agg2, x, batch3, W_nbr, W_root, b2)


# speedup vs baseline: 1.0030x; 1.0030x over previous
import jax
import jax.numpy as jnp
from jax import lax
from jax.experimental import pallas as pl
from jax.experimental.pallas import tpu as pltpu
from jax.experimental.pallas import tpu_sc as plsc

N = 10000
E = 320000
D = 128
G = 256

NC = 2
NS = 16
CH = 128
NCHUNKS = E // CH
COMMON = NCHUNKS // (NC * NS)
EXTRA = NCHUNKS - COMMON * NC * NS
NBUF = 3
RPT = 624


def _sc_body(x_hbm, ei_hbm, agg_hbm, *rest):
    ibuf = rest[:2 * NBUF]
    isem = rest[2 * NBUF:4 * NBUF]
    rows = rest[4 * NBUF:5 * NBUF]
    gsem = rest[5 * NBUF:6 * NBUF]
    ssem = rest[6 * NBUF:7 * NBUF]
    acc = rest[7 * NBUF]
    c = lax.axis_index("c")
    s = lax.axis_index("s")
    t = c * NS + s
    base = t * COMMON

    def _e(j):
        return pl.multiple_of((base + j) * CH, CH)

    def _i(j, p, b):
        return pltpu.make_async_copy(ei_hbm.at[:, pl.ds(_e(j), CH)],
                                     ibuf[p * NBUF + b], isem[p * NBUF + b])

    def _g(j, p, b):
        return pltpu.make_async_copy(x_hbm.at[ibuf[p * NBUF + b].at[0]],
                                     rows[b], gsem[b])

    def _s(j, p, b):
        return pltpu.make_async_copy(rows[b], acc.at[ibuf[p * NBUF + b].at[1]],
                                     ssem[b])

    for b in range(NBUF):
        _i(b, 0, b).start()

    def _zfill(i, carry):
        for b in range(NBUF):
            for j in range(D // 16):
                rows[b][i, pl.ds(j * 16, 16)] = jnp.zeros((16,), jnp.float32)
        return carry
    lax.fori_loop(0, CH, _zfill, 0)
    row0 = s * RPT
    for k in range(RPT // CH):
        pltpu.sync_copy(rows[k % NBUF], acc.at[pl.ds(row0 + k * CH, CH)])
    pltpu.sync_copy(rows[0].at[pl.ds(0, RPT - (RPT // CH) * CH)],
                    acc.at[pl.ds(row0 + (RPT // CH) * CH,
                                 RPT - (RPT // CH) * CH)])

    @pl.when(s == NS - 1)
    def _():
        left = N - NS * RPT
        pltpu.sync_copy(rows[0].at[pl.ds(0, left)],
                        acc.at[pl.ds(NS * RPT, left)])

    for b in range(NBUF):
        _i(b, 0, b).wait()
        _g(b, 0, b).start()
    plsc.subcore_barrier()

    def _one_group(g, par, prefetch):
        j0 = g * NBUF
        for b in range(NBUF):
            _g(j0 + b, par, b).wait()
            _s(j0 + b, par, b).start(add=True)
            if prefetch:
                _i(j0 + NBUF + b, 1 - par, b).start()
        for b in range(NBUF):
            _s(j0 + b, par, b).wait()
            if prefetch:
                _i(j0 + NBUF + b, 1 - par, b).wait()
                _g(j0 + NBUF + b, 1 - par, b).start()

    def _pair(k2, carry):
        _one_group(2 * k2, 0, True)
        _one_group(2 * k2 + 1, 1, True)
        return carry
    NG = COMMON // NBUF
    lax.fori_loop(0, NG // 2 - 1, _pair, 0)
    _one_group(NG - 2, 0, True)
    _one_group(NG - 1, 1, False)

    @pl.when(s < EXTRA // NC)
    def _():
        off = pl.multiple_of(
            (NC * NS * COMMON) * CH + (c * (EXTRA // NC) + s) * CH, CH)
        pltpu.sync_copy(ei_hbm.at[:, pl.ds(off, CH)], ibuf[0])
        _g(0, 0, 0).start()
        _g(0, 0, 0).wait()
        pltpu.sync_copy(rows[0], acc.at[ibuf[0].at[1]], add=True)

    plsc.subcore_barrier()

    pltpu.sync_copy(acc.at[pl.ds(row0, RPT)], agg_hbm.at[c, pl.ds(row0, RPT)])

    @pl.when(s == NS - 1)
    def _():
        left = N - NS * RPT
        pltpu.sync_copy(acc.at[pl.ds(NS * RPT, left)],
                        agg_hbm.at[c, pl.ds(NS * RPT, left)])


@jax.jit
def _sc_scatter(x, edge_index):
    mesh = plsc.VectorSubcoreMesh(core_axis_name="c", subcore_axis_name="s")
    scratch = [pltpu.VMEM((2, CH), jnp.int32) for _ in range(2 * NBUF)]
    scratch += [pltpu.SemaphoreType.DMA for _ in range(2 * NBUF)]
    scratch += [pltpu.VMEM((CH, D), jnp.float32) for _ in range(NBUF)]
    scratch += [pltpu.SemaphoreType.DMA for _ in range(2 * NBUF)]
    scratch += [pltpu.MemorySpace.VMEM_SHARED((N, D), jnp.float32)]
    return pl.kernel(
        _sc_body,
        out_type=jax.ShapeDtypeStruct((NC, N, D), jnp.float32),
        mesh=mesh,
        scratch_types=scratch,
    )(x, edge_index)


BLK = 2000
NBLK = N // BLK


def _tc_body(agg_ref, x_ref, batch_ref, wn_ref, wr_ref, b_ref, out_ref):
    i = pl.program_id(0)
    a = (agg_ref[0] + agg_ref[1]).astype(jnp.bfloat16)
    h = jnp.dot(a, wn_ref[...].astype(jnp.bfloat16),
                preferred_element_type=jnp.float32)
    h = h + jnp.dot(x_ref[...].astype(jnp.bfloat16),
                    wr_ref[...].astype(jnp.bfloat16),
                    preferred_element_type=jnp.float32)
    h = jnp.maximum(h + b_ref[...], 0.0)
    bt = batch_ref[0, 0, :]
    gid = lax.broadcasted_iota(jnp.int32, (G, BLK), 0)
    onehot = jnp.where(gid == bt[None, :], 1.0, 0.0).astype(jnp.bfloat16)
    p = jnp.dot(onehot, h.astype(jnp.bfloat16),
                preferred_element_type=jnp.float32)

    @pl.when(i == 0)
    def _():
        out_ref[...] = p

    @pl.when(i > 0)
    def _():
        out_ref[...] += p


@jax.jit
def _tc_combine(agg2, x, batch3, W_nbr, W_root, b2):
    return pl.pallas_call(
        _tc_body,
        grid=(NBLK,),
        in_specs=[
            pl.BlockSpec((NC, BLK, D), lambda i: (0, i, 0)),
            pl.BlockSpec((BLK, D), lambda i: (i, 0)),
            pl.BlockSpec((1, 1, BLK), lambda i: (i, 0, 0)),
            pl.BlockSpec((D, D), lambda i: (0, 0)),
            pl.BlockSpec((D, D), lambda i: (0, 0)),
            pl.BlockSpec((1, D), lambda i: (0, 0)),
        ],
        out_specs=pl.BlockSpec((G, D), lambda i: (0, 0)),
        out_shape=jax.ShapeDtypeStruct((G, D), jnp.float32),
    )(agg2, x, batch3, W_nbr, W_root, b2)


def kernel(x, edge_index, batch, W_nbr, W_root, b):
    agg2 = _sc_scatter(x, edge_index)
    batch3 = batch.reshape(NBLK, 1, BLK)
    b2 = b.reshape(1, D)
    return _tc_combine(agg2, x, batch3, W_nbr, W_root, b2)
